# baseline (speedup 1.0000x reference)
import functools

import jax
import jax.numpy as jnp
from jax import lax
from jax.experimental import pallas as pl
from jax.experimental.pallas import tpu as pltpu
from jax.experimental.pallas import tpu_sc as plsc

NOBJ = 40
NSEG = 48
NBLK = 8192
NEG = -3.0e38
NC = 2
NS = 16
NW = NC * NS
GG = 256


def _tile_kernel(params_ref, pts_ref, feat_ref, pf_ref, ids_ref, *, nobj):
    feats = feat_ref[0]
    pf_ref[...] = feats.T

    x = pts_ref[0, 1:2, :]
    y = pts_ref[0, 2:3, :]
    z = pts_ref[0, 3:4, :]
    bx = params_ref[0]
    pad = jnp.zeros((NSEG - nobj, 1), bx.dtype)
    npad = jnp.full((NSEG - nobj, 1), -1.0, bx.dtype)
    cx = jnp.concatenate([bx[:, 0:1], pad])
    cy = jnp.concatenate([bx[:, 1:2], pad])
    cz = jnp.concatenate([bx[:, 2:3], pad])
    hx = jnp.concatenate([bx[:, 3:4] * 0.5, npad])
    hy = jnp.concatenate([bx[:, 4:5] * 0.5, npad])
    hz = jnp.concatenate([bx[:, 5:6] * 0.5, npad])
    ca = jnp.concatenate([jnp.cos(-bx[:, 6:7]), pad])
    sa = jnp.concatenate([jnp.sin(-bx[:, 6:7]), pad])

    sx = x - cx
    sy = y - cy
    sz = z - cz
    lx = sx * ca - sy * sa
    ly = sx * sa + sy * ca
    inb = (jnp.abs(lx) <= hx) & (jnp.abs(ly) <= hy) & (jnp.abs(sz) <= hz)
    bi = jax.lax.broadcasted_iota(jnp.int32, (NSEG, NBLK), 0)
    sel = jnp.min(jnp.where(inb, bi, NOBJ), axis=0, keepdims=True)
    ids_ref[...] = sel.reshape(1, 1, NBLK)


def _seg_kernel(pf_hbm, ids_hbm, out_hbm, ibuf, cpak, fbufa, fbufb,
                acca, accb, shared, mbuf, tbuf, sema, semb, *, ppw, nobj):
    c = 128
    cid = lax.axis_index("c")
    sid = lax.axis_index("s")
    wid = cid * NS + sid
    base = wid * ppw
    nchunks = ppw // GG

    negv = jnp.full((16,), NEG, jnp.float32)
    bgv = jnp.full((16,), NOBJ * c, jnp.int32)

    def init_acc(i, _):
        acca[pl.ds(i * 16, 16)] = negv
        accb[pl.ds(i * 16, 16)] = negv
        return 0
    lax.fori_loop(0, NSEG * c // 16, init_acc, 0)

    pltpu.sync_copy(ids_hbm.at[pl.ds(base, ppw)], ibuf)

    def start(g, buf, sem):
        pltpu.make_async_copy(pf_hbm.at[pl.ds(base + g * GG, GG)], buf,
                              sem).start()

    def wait(g, buf, sem):
        pltpu.make_async_copy(pf_hbm.at[pl.ds(base + g * GG, GG)], buf,
                              sem).wait()

    def process(g, fbuf):
        def compact(q, cur):
            idsv = ibuf[pl.ds(g * GG + q * 16, 16)]
            mask = idsv < NOBJ
            pak = idsv * c + (lax.iota(jnp.int32, 16) + q * 16) * 16384
            plsc.store_compressed(cpak.at[pl.ds(cur, 16)], pak, mask=mask)
            return cur + plsc.all_reduce_population_count(mask)[0]
        cnt = lax.fori_loop(0, GG // 16, compact, jnp.int32(0))
        cpak[pl.ds(cnt, 16)] = bgv
        ngroups = (cnt + 15) // 16

        def update(q, _):
            pakv = cpak[pl.ds(q * 16, 16)]
            for j in range(16):
                acc = acca if j % 2 == 0 else accb
                pak = pakv[j]
                off = lax.rem(pak, 16384)
                lrow = lax.div(pak, 16384)
                for v in range(8):
                    a = acc[pl.ds(off + v * 16, 16)]
                    f = fbuf[lrow, pl.ds(v * 16, 16)]
                    acc[pl.ds(off + v * 16, 16)] = jnp.maximum(a, f)
            return 0
        lax.fori_loop(0, ngroups, update, 0)

    start(0, fbufa, sema)
    for t in range(nchunks // 2):
        wait(2 * t, fbufa, sema)
        start(2 * t + 1, fbufb, semb)
        process(2 * t, fbufa)
        wait(2 * t + 1, fbufb, semb)
        if t + 1 < nchunks // 2:
            start(2 * t + 2, fbufa, sema)
        process(2 * t + 1, fbufb)

    def bmerge(i, _):
        acca[pl.ds(i * 16, 16)] = jnp.maximum(acca[pl.ds(i * 16, 16)],
                                              accb[pl.ds(i * 16, 16)])
        return 0
    lax.fori_loop(0, NSEG * c // 16, bmerge, 0)
    pltpu.sync_copy(acca, shared.at[sid])
    plsc.subcore_barrier()

    width = nobj * c // 8
    scene_local = sid // 8
    part = lax.rem(sid, 8)
    col0 = part * width
    pltpu.sync_copy(shared.at[scene_local * 8, pl.ds(col0, width)], mbuf)

    def tmerge(r, _):
        pltpu.sync_copy(shared.at[scene_local * 8 + r, pl.ds(col0, width)],
                        tbuf)

        def vv(i, _):
            mbuf[pl.ds(i * 16, 16)] = jnp.maximum(mbuf[pl.ds(i * 16, 16)],
                                                  tbuf[pl.ds(i * 16, 16)])
            return 0
        lax.fori_loop(0, width // 16, vv, 0)
        return 0
    lax.fori_loop(1, 8, tmerge, 0)

    def sfix(i, _):
        v = mbuf[pl.ds(i * 16, 16)]
        mbuf[pl.ds(i * 16, 16)] = jnp.where(v < -1.0e38, 0.0, v)
        return 0
    lax.fori_loop(0, width // 16, sfix, 0)

    scene = cid * 2 + scene_local
    pltpu.sync_copy(mbuf, out_hbm.at[scene, 0, pl.ds(col0, width)])


def kernel(point_features, points, gt_boxes, batch_size):
    bs, c, n_per = point_features.shape
    nobj = gt_boxes.shape[1]
    k_blocks = n_per // NBLK
    ppw = bs * n_per // NW
    tecs_per_scene = NW // bs

    pts_t = points.reshape(bs, n_per, 4).transpose(0, 2, 1)

    pf, ids3 = pl.pallas_call(
        functools.partial(_tile_kernel, nobj=nobj),
        grid=(bs, k_blocks),
        in_specs=[
            pl.BlockSpec((1, nobj, 8), lambda b, k: (b, 0, 0)),
            pl.BlockSpec((1, 4, NBLK), lambda b, k: (b, 0, k)),
            pl.BlockSpec((1, c, NBLK), lambda b, k: (b, 0, k)),
        ],
        out_specs=[
            pl.BlockSpec((NBLK, c), lambda b, k: (b * k_blocks + k, 0)),
            pl.BlockSpec((1, 1, NBLK), lambda b, k: (b * k_blocks + k, 0, 0)),
        ],
        out_shape=[
            jax.ShapeDtypeStruct((bs * n_per, c), point_features.dtype),
            jax.ShapeDtypeStruct((bs * k_blocks, 1, NBLK), jnp.int32),
        ],
    )(gt_boxes, pts_t, point_features)
    ids = ids3.reshape(bs * n_per)

    mesh = plsc.VectorSubcoreMesh(core_axis_name="c", subcore_axis_name="s")
    seg_fn = functools.partial(
        pl.kernel,
        mesh=mesh,
        out_type=jax.ShapeDtypeStruct((bs, 1, nobj * c), jnp.float32),
        scratch_types=[
            pltpu.VMEM((ppw,), jnp.int32),
            pltpu.VMEM((GG + 16,), jnp.int32),
            pltpu.VMEM((GG, c), jnp.float32),
            pltpu.VMEM((GG, c), jnp.float32),
            pltpu.VMEM((NSEG * c,), jnp.float32),
            pltpu.VMEM((NSEG * c,), jnp.float32),
            pltpu.VMEM_SHARED((NS, NSEG * c), jnp.float32),
            pltpu.VMEM((nobj * c // 8,), jnp.float32),
            pltpu.VMEM((nobj * c // 8,), jnp.float32),
            pltpu.SemaphoreType.DMA,
            pltpu.SemaphoreType.DMA,
        ],
        compiler_params=pltpu.CompilerParams(needs_layout_passes=False),
    )(functools.partial(_seg_kernel, ppw=ppw, nobj=nobj))
    seg = seg_fn(pf, ids)

    all_seg = seg.reshape(bs * nobj, c)
    return all_seg, pf
